# Initial kernel scaffold; baseline (speedup 1.0000x reference)
import jax
import jax.numpy as jnp
from jax.experimental import pallas as pl


def kernel(gene_emb, global_ids, edge_index, W1, b1, W2, b2):
    raise NotImplementedError("write your pallas kernel here")


# speedup vs baseline: 43.0037x; 43.0037x over previous
import functools

import jax
import jax.numpy as jnp
from jax import lax
from jax.experimental import pallas as pl
from jax.experimental.pallas import tpu as pltpu
from jax.experimental.pallas import tpu_sc as plsc

B = 4
N = 10000
NP = 10240
G = 20000
D = 128
E = 160000
EP = 163840
NC, NS = 2, 16
NW = NC * NS
EPT = EP // NW
ECH = EPT // 128
RPW = (NP * B) // NW
RCH = RPW // 128
SLICE = NP // NS


def _fill1d(ref, n, val):
    v = jnp.full((16,), val, jnp.float32)
    for i in range(n // 16):
        ref[pl.ds(i * 16, 16)] = v


def _fill2d(ref, rows, val):
    v = jnp.full((16,), val, jnp.float32)
    for r in range(rows):
        for u in range(D // 16):
            ref[r, pl.ds(u * 16, 16)] = v


def _kg_body(gene, idxg, dst3, x_out, degp,
             degs, idxv, dstv, gb0, gb1, zv, onesv, sem0, sem1):
    c = lax.axis_index("c")
    s = lax.axis_index("s")
    w = c * NS + s
    b = w // 8
    k = w % 8
    _fill1d(zv, SLICE, 0.0)
    _fill1d(onesv, 128, 1.0)
    pltpu.sync_copy(zv, degs.at[pl.ds(s * SLICE, SLICE)])
    plsc.subcore_barrier()
    pltpu.sync_copy(dst3.at[w], dstv)
    for j in range(ECH):
        pltpu.sync_copy(onesv, degs.at[dstv.at[j]], add=True)
    pltpu.sync_copy(idxg.at[b].at[k], idxv)
    bufs = (gb0, gb1)
    sems = (sem0, sem1)
    cps = [pltpu.async_copy(gene.at[idxv.at[pl.ds(0, 128)]], gb0, sem0), None]
    for j in range(RCH):
        cur = j % 2
        nxt = 1 - cur
        if j + 1 < RCH:
            cps[nxt] = pltpu.async_copy(
                gene.at[idxv.at[pl.ds((j + 1) * 128, 128)]], bufs[nxt], sems[nxt])
        cps[cur].wait()
        pltpu.sync_copy(bufs[cur], x_out.at[b].at[pl.ds(k * RPW + j * 128, 128)])
    plsc.subcore_barrier()
    pltpu.sync_copy(degs.at[pl.ds(s * SLICE, SLICE)],
                    degp.at[c].at[pl.ds(s * SLICE, SLICE)])


_kg = functools.partial(
    pl.kernel,
    out_type=(jax.ShapeDtypeStruct((B, NP, D), jnp.float32),
              jax.ShapeDtypeStruct((NC, NP), jnp.float32)),
    mesh=plsc.VectorSubcoreMesh(core_axis_name="c", subcore_axis_name="s"),
    scratch_types=[
        pltpu.VMEM_SHARED((NP,), jnp.float32),
        pltpu.VMEM((RPW,), jnp.int32),
        pltpu.VMEM((ECH, 128), jnp.int32),
        pltpu.VMEM((128, D), jnp.float32),
        pltpu.VMEM((128, D), jnp.float32),
        pltpu.VMEM((SLICE,), jnp.float32),
        pltpu.VMEM((128,), jnp.float32),
        pltpu.SemaphoreType.DMA,
        pltpu.SemaphoreType.DMA,
    ],
)(_kg_body)


def _k2_body(y, src3, dst3, dis, spart, cep,
             S, ce, srcv, dstv, vb0, vb1, gb0, gb1, zb, zv, sem0, sem1):
    c = lax.axis_index("c")
    s = lax.axis_index("s")
    w = c * NS + s
    _fill1d(zv, SLICE, 0.0)
    _fill2d(zb, 16, 0.0)
    pltpu.sync_copy(src3.at[w], srcv)
    pltpu.sync_copy(dst3.at[w], dstv)
    pltpu.sync_copy(zv, ce.at[pl.ds(s * SLICE, SLICE)])
    plsc.subcore_barrier()
    vbufs = (vb0, vb1)
    vsems = (sem0, sem1)
    vps = [pltpu.async_copy(dis.at[0].at[dstv.at[0]], vb0, sem0), None]
    for j in range(ECH):
        cur = j % 2
        nxt = 1 - cur
        if j + 1 < ECH:
            vps[nxt] = pltpu.async_copy(dis.at[0].at[dstv.at[j + 1]],
                                        vbufs[nxt], vsems[nxt])
        vps[cur].wait()
        pltpu.sync_copy(vbufs[cur], ce.at[srcv.at[j]], add=True)
    plsc.subcore_barrier()
    pltpu.sync_copy(ce.at[pl.ds(s * SLICE, SLICE)],
                    cep.at[c].at[pl.ds(s * SLICE, SLICE)])
    bufs = (gb0, gb1)
    sems = (sem0, sem1)
    for b in range(B):
        for q in range(SLICE // 16):
            pltpu.sync_copy(zb, S.at[pl.ds(s * SLICE + q * 16, 16)])
        plsc.subcore_barrier()
        cps = [pltpu.async_copy(y.at[b].at[srcv.at[0]], gb0, sem0), None]
        for j in range(ECH):
            cur = j % 2
            nxt = 1 - cur
            if j + 1 < ECH:
                cps[nxt] = pltpu.async_copy(y.at[b].at[srcv.at[j + 1]],
                                            bufs[nxt], sems[nxt])
            cps[cur].wait()
            pltpu.sync_copy(bufs[cur], S.at[dstv.at[j]], add=True)
        plsc.subcore_barrier()
        for q in range(SLICE // 128):
            pltpu.sync_copy(
                S.at[pl.ds(s * SLICE + q * 128, 128)],
                spart.at[b].at[c].at[pl.ds(s * SLICE + q * 128, 128)])


_k2 = functools.partial(
    pl.kernel,
    out_type=(jax.ShapeDtypeStruct((B, NC, NP, D), jnp.float32),
              jax.ShapeDtypeStruct((NC, NP), jnp.float32)),
    mesh=plsc.VectorSubcoreMesh(core_axis_name="c", subcore_axis_name="s"),
    scratch_types=[
        pltpu.VMEM_SHARED((NP, D), jnp.float32),
        pltpu.VMEM_SHARED((NP,), jnp.float32),
        pltpu.VMEM((ECH, 128), jnp.int32),
        pltpu.VMEM((ECH, 128), jnp.int32),
        pltpu.VMEM((128,), jnp.float32),
        pltpu.VMEM((128,), jnp.float32),
        pltpu.VMEM((128, D), jnp.float32),
        pltpu.VMEM((128, D), jnp.float32),
        pltpu.VMEM((16, D), jnp.float32),
        pltpu.VMEM((SLICE,), jnp.float32),
        pltpu.SemaphoreType.DMA,
        pltpu.SemaphoreType.DMA,
    ],
)(_k2_body)


def _t1_body(degp_ref, x_ref, w1_ref, y_ref, dis_ref):
    deg = degp_ref[0] + degp_ref[1] + 1.0
    dis = lax.rsqrt(deg)
    xw = jnp.dot(x_ref[0], w1_ref[...], precision=lax.Precision.HIGHEST,
                 preferred_element_type=jnp.float32)
    y_ref[0] = dis[:, None] * xw
    dis_ref[0] = dis


def _t1(degp, x, W1):
    blk = 1024
    return pl.pallas_call(
        _t1_body,
        grid=(B, NP // blk),
        in_specs=[
            pl.BlockSpec((NC, blk), lambda b, i: (0, i)),
            pl.BlockSpec((1, blk, D), lambda b, i: (b, i, 0)),
            pl.BlockSpec((D, D), lambda b, i: (0, 0)),
        ],
        out_specs=[
            pl.BlockSpec((1, blk, D), lambda b, i: (b, i, 0)),
            pl.BlockSpec((1, blk), lambda b, i: (0, i)),
        ],
        out_shape=[jax.ShapeDtypeStruct((B, NP, D), jnp.float32),
                   jax.ShapeDtypeStruct((1, NP), jnp.float32)],
    )(degp, x, W1)


def _t2_body(s_ref, y_ref, dis_ref, cep_ref, b1_ref, w2_ref, b2_ref,
             out_ref, acc_ref):
    blk = s_ref.shape[2]
    i = pl.program_id(1)
    disv = dis_ref[0]
    pre = disv[:, None] * (s_ref[0, 0] + s_ref[0, 1] + y_ref[0]) \
        + b1_ref[0][None, :]
    h = jnp.maximum(pre, 0.0)
    rows = i * blk + lax.broadcasted_iota(jnp.int32, (blk,), 0)
    ce = cep_ref[0] + cep_ref[1]
    cvec = jnp.where(rows < N, disv * (disv + ce), 0.0)
    part = lax.dot_general(cvec[None, :], h, (((1,), (0,)), ((), ())),
                           precision=lax.Precision.HIGHEST,
                           preferred_element_type=jnp.float32)

    @pl.when(i == 0)
    def _():
        acc_ref[...] = part

    @pl.when(i > 0)
    def _():
        acc_ref[...] += part

    @pl.when(i == pl.num_programs(1) - 1)
    def _():
        b = pl.program_id(0)
        out_ref[pl.ds(b, 1), :] = jnp.dot(acc_ref[...] * (1.0 / N), w2_ref[...],
                                          precision=lax.Precision.HIGHEST,
                                          preferred_element_type=jnp.float32) \
            + b2_ref[...]


def _t2(spart, y, dis, cep, b1, W2, b2):
    blk = 1024
    return pl.pallas_call(
        _t2_body,
        grid=(B, NP // blk),
        in_specs=[
            pl.BlockSpec((1, NC, blk, D), lambda b, i: (b, 0, i, 0)),
            pl.BlockSpec((1, blk, D), lambda b, i: (b, i, 0)),
            pl.BlockSpec((1, blk), lambda b, i: (0, i)),
            pl.BlockSpec((NC, blk), lambda b, i: (0, i)),
            pl.BlockSpec((1, D), lambda b, i: (0, 0)),
            pl.BlockSpec((D, D), lambda b, i: (0, 0)),
            pl.BlockSpec((1, D), lambda b, i: (0, 0)),
        ],
        out_specs=pl.BlockSpec((B, D), lambda b, i: (0, 0)),
        out_shape=jax.ShapeDtypeStruct((B, D), jnp.float32),
        scratch_shapes=[pltpu.VMEM((1, D), jnp.float32)],
    )(spart, y, dis, cep, b1, W2, b2)


def kernel(gene_emb, global_ids, edge_index, W1, b1, W2, b2):
    gene = gene_emb.reshape(B * G, D)
    gid = jnp.concatenate(
        [global_ids.astype(jnp.int32), jnp.zeros((NP - N,), jnp.int32)])
    offs = (jnp.arange(B, dtype=jnp.int32) * G)[:, None]
    idxg = (gid[None, :] + offs).reshape(B, 8, RPW)
    pad = (N + (jnp.arange(EP - E, dtype=jnp.int32) % (NP - N))).astype(jnp.int32)
    src3 = jnp.concatenate([edge_index[0].astype(jnp.int32), pad]).reshape(NW, ECH, 128)
    dst3 = jnp.concatenate([edge_index[1].astype(jnp.int32), pad]).reshape(NW, ECH, 128)

    x, degp = _kg(gene, idxg, dst3)
    y, dis = _t1(degp, x, W1)
    spart, cep = _k2(y, src3, dst3, dis)
    return _t2(spart, y, dis, cep, b1.reshape(1, D), W2, b2.reshape(1, D))


# speedup vs baseline: 43.4277x; 1.0099x over previous
import functools

import jax
import jax.numpy as jnp
from jax import lax
from jax.experimental import pallas as pl
from jax.experimental.pallas import tpu as pltpu
from jax.experimental.pallas import tpu_sc as plsc

B = 4
N = 10000
NP = 10240
G = 20000
D = 128
E = 160000
EP = 163840
NC, NS = 2, 16
NW = NC * NS
ECT = EP // NS
ECH = ECT // 128
CCH = ECH // NC
EROW = 64
KCH = ECT // EROW
KCC = KCH // NC
RPW = (NP * B) // NW
RCH = RPW // 128
SLICE = NP // NS


def _fill1d(ref, n, val):
    v = jnp.full((16,), val, jnp.float32)
    for i in range(n // 16):
        ref[pl.ds(i * 16, 16)] = v


def _fill2d(ref, rows, val):
    v = jnp.full((16,), val, jnp.float32)
    for r in range(rows):
        for u in range(D // 16):
            ref[r, pl.ds(u * 16, 16)] = v


def _kg_body(gene, idxg, dst3, x_out, degp,
             degs, idxv, dstv, gb0, gb1, zv, onesv, sem0, sem1, sem2):
    c = lax.axis_index("c")
    s = lax.axis_index("s")
    w = c * NS + s
    b = w // 8
    k = w % 8
    _fill1d(zv, SLICE, 0.0)
    _fill1d(onesv, 128, 1.0)
    pltpu.sync_copy(zv, degs.at[pl.ds(s * SLICE, SLICE)])
    plsc.subcore_barrier()
    pltpu.sync_copy(dst3.at[w], dstv)
    dps = [pltpu.async_copy(onesv, degs.at[dstv.at[j]], sem2, add=True)
           for j in range(CCH)]
    pltpu.sync_copy(idxg.at[b].at[k], idxv)
    bufs = (gb0, gb1)
    sems = (sem0, sem1)
    cps = [pltpu.async_copy(gene.at[idxv.at[pl.ds(0, 128)]], gb0, sem0), None]
    for j in range(RCH):
        cur = j % 2
        nxt = 1 - cur
        if j + 1 < RCH:
            cps[nxt] = pltpu.async_copy(
                gene.at[idxv.at[pl.ds((j + 1) * 128, 128)]], bufs[nxt], sems[nxt])
        cps[cur].wait()
        pltpu.sync_copy(bufs[cur], x_out.at[b].at[pl.ds(k * RPW + j * 128, 128)])
    for dp in dps:
        dp.wait()
    plsc.subcore_barrier()
    pltpu.sync_copy(degs.at[pl.ds(s * SLICE, SLICE)],
                    degp.at[c].at[pl.ds(s * SLICE, SLICE)])


_kg = functools.partial(
    pl.kernel,
    out_type=(jax.ShapeDtypeStruct((B, NP, D), jnp.float32),
              jax.ShapeDtypeStruct((NC, NP), jnp.float32)),
    mesh=plsc.VectorSubcoreMesh(core_axis_name="c", subcore_axis_name="s"),
    scratch_types=[
        pltpu.VMEM_SHARED((NP,), jnp.float32),
        pltpu.VMEM((RPW,), jnp.int32),
        pltpu.VMEM((CCH, 128), jnp.int32),
        pltpu.VMEM((128, D), jnp.float32),
        pltpu.VMEM((128, D), jnp.float32),
        pltpu.VMEM((SLICE,), jnp.float32),
        pltpu.VMEM((128,), jnp.float32),
        pltpu.SemaphoreType.DMA,
        pltpu.SemaphoreType.DMA,
        pltpu.SemaphoreType.DMA,
    ],
)(_kg_body)


def _k2_body(y, src2, dst2, dis, sout, cep,
             S, ce, srcv, dstv, vb0, vb1, gb0, gb1, zb, zv,
             sem0, sem1, ss0, ss1, cgs, css):
    c = lax.axis_index("c")
    s = lax.axis_index("s")
    _fill1d(zv, SLICE, 0.0)
    _fill2d(zb, 16, 0.0)
    pltpu.sync_copy(src2.at[s], srcv)
    pltpu.sync_copy(dst2.at[s], dstv)
    pltpu.sync_copy(zv, ce.at[pl.ds(s * SLICE, SLICE)])
    plsc.subcore_barrier()
    j0 = c * KCC
    vbufs = (vb0, vb1)
    cgp = [pltpu.async_copy(
        dis.at[0].at[dstv.at[pl.ds((j0 + j) * EROW, EROW)]], vbufs[j], cgs)
        for j in range(2)]
    csp = [None, None]
    bufs = (gb0, gb1)
    sems = (sem0, sem1)
    ssems = (ss0, ss1)
    for bb in range(B // NC):
        b = c * (B // NC) + bb
        for q in range(SLICE // 16):
            pltpu.sync_copy(zb, S.at[pl.ds(s * SLICE + q * 16, 16)])
        plsc.subcore_barrier()
        gps = [pltpu.async_copy(
            y.at[b].at[srcv.at[pl.ds(0, EROW)]], gb0, sem0), None]
        sps = [None, None]
        for j in range(KCH):
            cur = j % 2
            nxt = 1 - cur
            if j + 1 < KCH:
                if sps[nxt] is not None:
                    sps[nxt].wait()
                    sps[nxt] = None
                gps[nxt] = pltpu.async_copy(
                    y.at[b].at[srcv.at[pl.ds((j + 1) * EROW, EROW)]],
                    bufs[nxt], sems[nxt])
            gps[cur].wait()
            sps[cur] = pltpu.async_copy(
                bufs[cur], S.at[dstv.at[pl.ds(j * EROW, EROW)]],
                ssems[cur], add=True)
            if bb == 0 and j < KCC:
                p = j % 2
                cgp[p].wait()
                csp[p] = pltpu.async_copy(
                    vbufs[p], ce.at[srcv.at[pl.ds((j0 + j) * EROW, EROW)]],
                    css, add=True)
                if j + 2 < KCC:
                    csp[p].wait()
                    csp[p] = None
                    cgp[p] = pltpu.async_copy(
                        dis.at[0].at[dstv.at[pl.ds((j0 + j + 2) * EROW, EROW)]],
                        vbufs[p], cgs)
        for sp in sps:
            if sp is not None:
                sp.wait()
        plsc.subcore_barrier()
        for q in range(SLICE // 128):
            pltpu.sync_copy(
                S.at[pl.ds(s * SLICE + q * 128, 128)],
                sout.at[b].at[pl.ds(s * SLICE + q * 128, 128)])
    for p in csp:
        if p is not None:
            p.wait()
    plsc.subcore_barrier()
    pltpu.sync_copy(ce.at[pl.ds(s * SLICE, SLICE)],
                    cep.at[c].at[pl.ds(s * SLICE, SLICE)])


_k2 = functools.partial(
    pl.kernel,
    out_type=(jax.ShapeDtypeStruct((B, NP, D), jnp.float32),
              jax.ShapeDtypeStruct((NC, NP), jnp.float32)),
    mesh=plsc.VectorSubcoreMesh(core_axis_name="c", subcore_axis_name="s"),
    scratch_types=[
        pltpu.VMEM_SHARED((NP, D), jnp.float32),
        pltpu.VMEM_SHARED((NP,), jnp.float32),
        pltpu.VMEM((ECT,), jnp.int32),
        pltpu.VMEM((ECT,), jnp.int32),
        pltpu.VMEM((EROW,), jnp.float32),
        pltpu.VMEM((EROW,), jnp.float32),
        pltpu.VMEM((EROW, D), jnp.float32),
        pltpu.VMEM((EROW, D), jnp.float32),
        pltpu.VMEM((16, D), jnp.float32),
        pltpu.VMEM((SLICE,), jnp.float32),
        pltpu.SemaphoreType.DMA,
        pltpu.SemaphoreType.DMA,
        pltpu.SemaphoreType.DMA,
        pltpu.SemaphoreType.DMA,
        pltpu.SemaphoreType.DMA,
        pltpu.SemaphoreType.DMA,
    ],
)(_k2_body)


def _t1_body(degp_ref, x_ref, w1_ref, y_ref, dis_ref):
    deg = degp_ref[0] + degp_ref[1] + 1.0
    dis = lax.rsqrt(deg)
    xw = jnp.dot(x_ref[0], w1_ref[...], precision=lax.Precision.HIGHEST,
                 preferred_element_type=jnp.float32)
    y_ref[0] = dis[:, None] * xw
    dis_ref[0] = dis


def _t1(degp, x, W1):
    blk = 1024
    return pl.pallas_call(
        _t1_body,
        grid=(B, NP // blk),
        in_specs=[
            pl.BlockSpec((NC, blk), lambda b, i: (0, i)),
            pl.BlockSpec((1, blk, D), lambda b, i: (b, i, 0)),
            pl.BlockSpec((D, D), lambda b, i: (0, 0)),
        ],
        out_specs=[
            pl.BlockSpec((1, blk, D), lambda b, i: (b, i, 0)),
            pl.BlockSpec((1, blk), lambda b, i: (0, i)),
        ],
        out_shape=[jax.ShapeDtypeStruct((B, NP, D), jnp.float32),
                   jax.ShapeDtypeStruct((1, NP), jnp.float32)],
    )(degp, x, W1)


def _t2_body(s_ref, y_ref, dis_ref, cep_ref, b1_ref, w2_ref, b2_ref,
             out_ref, acc_ref):
    blk = s_ref.shape[1]
    i = pl.program_id(1)
    disv = dis_ref[0]
    pre = disv[:, None] * (s_ref[0] + y_ref[0]) + b1_ref[0][None, :]
    h = jnp.maximum(pre, 0.0)
    rows = i * blk + lax.broadcasted_iota(jnp.int32, (blk,), 0)
    ce = cep_ref[0] + cep_ref[1]
    cvec = jnp.where(rows < N, disv * (disv + ce), 0.0)
    part = lax.dot_general(cvec[None, :], h, (((1,), (0,)), ((), ())),
                           precision=lax.Precision.HIGHEST,
                           preferred_element_type=jnp.float32)

    @pl.when(i == 0)
    def _():
        acc_ref[...] = part

    @pl.when(i > 0)
    def _():
        acc_ref[...] += part

    @pl.when(i == pl.num_programs(1) - 1)
    def _():
        b = pl.program_id(0)
        out_ref[pl.ds(b, 1), :] = jnp.dot(acc_ref[...] * (1.0 / N), w2_ref[...],
                                          precision=lax.Precision.HIGHEST,
                                          preferred_element_type=jnp.float32) \
            + b2_ref[...]


def _t2(sout, y, dis, cep, b1, W2, b2):
    blk = 1024
    return pl.pallas_call(
        _t2_body,
        grid=(B, NP // blk),
        in_specs=[
            pl.BlockSpec((1, blk, D), lambda b, i: (b, i, 0)),
            pl.BlockSpec((1, blk, D), lambda b, i: (b, i, 0)),
            pl.BlockSpec((1, blk), lambda b, i: (0, i)),
            pl.BlockSpec((NC, blk), lambda b, i: (0, i)),
            pl.BlockSpec((1, D), lambda b, i: (0, 0)),
            pl.BlockSpec((D, D), lambda b, i: (0, 0)),
            pl.BlockSpec((1, D), lambda b, i: (0, 0)),
        ],
        out_specs=pl.BlockSpec((B, D), lambda b, i: (0, 0)),
        out_shape=jax.ShapeDtypeStruct((B, D), jnp.float32),
        scratch_shapes=[pltpu.VMEM((1, D), jnp.float32)],
    )(sout, y, dis, cep, b1, W2, b2)


def kernel(gene_emb, global_ids, edge_index, W1, b1, W2, b2):
    gene = gene_emb.reshape(B * G, D)
    gid = jnp.concatenate(
        [global_ids.astype(jnp.int32), jnp.zeros((NP - N,), jnp.int32)])
    offs = (jnp.arange(B, dtype=jnp.int32) * G)[:, None]
    idxg = (gid[None, :] + offs).reshape(B, 8, RPW)
    pad = (N + (jnp.arange(EP - E, dtype=jnp.int32) % (NP - N))).astype(jnp.int32)
    srcp = jnp.concatenate([edge_index[0].astype(jnp.int32), pad])
    dstp = jnp.concatenate([edge_index[1].astype(jnp.int32), pad])
    src3 = srcp.reshape(NW, CCH, 128)
    dst3 = dstp.reshape(NW, CCH, 128)
    src2 = srcp.reshape(NS, ECT)
    dst2 = dstp.reshape(NS, ECT)

    x, degp = _kg(gene, idxg, dst3)
    y, dis = _t1(degp, x, W1)
    sout, cep = _k2(y, src2, dst2, dis)
    return _t2(sout, y, dis, cep, b1.reshape(1, D), W2, b2.reshape(1, D))
